# Initial kernel scaffold; baseline (speedup 1.0000x reference)
import jax
import jax.numpy as jnp
from jax.experimental import pallas as pl


def kernel(x, type_batch_idx, w_omega, b_omega, u_omega):
    raise NotImplementedError("write your pallas kernel here")


# speedup vs baseline: 4.9085x; 4.9085x over previous
import jax
import jax.numpy as jnp
from jax import lax
from jax.experimental import pallas as pl
from jax.experimental.pallas import tpu as pltpu
from jax.experimental.pallas import tpu_sc as plsc

N = 320000
EMB = 128
NSEG = 10000
SEG_PAD = 10240
DN_PAD = 12288

NC = 2
NS = 16
NW = NC * NS
LANES = 16

ROWS_PER_TILE = N // NW
CHUNK = 80
NCHUNKS = ROWS_PER_TILE // CHUNK
WB = 32

SEG_PER_TILE = SEG_PAD // NS
OUT_PER_TILE = SEG_PAD // NW
DN_SLICE = DN_PAD // NS
A_CHUNK = 400
A_NCHUNKS = ROWS_PER_TILE // A_CHUNK

TC_BLK = 1024
TC_GRID = (N + TC_BLK - 1) // TC_BLK
E_ROWS = TC_GRID * (TC_BLK // 128)


def _e_body(x_ref, w_ref, b_ref, u_ref, e_ref):
    v = jnp.tanh(
        jnp.dot(x_ref[...], w_ref[...], preferred_element_type=jnp.float32)
        + b_ref[...]
    )
    vu = jnp.sum(v * u_ref[...], axis=1)
    e_ref[...] = jnp.exp(vu).reshape(TC_BLK // 128, 128)


def _compute_e(x, w, b_row, u_row):
    return pl.pallas_call(
        _e_body,
        grid=(TC_GRID,),
        in_specs=[
            pl.BlockSpec((TC_BLK, EMB), lambda i: (i, 0)),
            pl.BlockSpec((EMB, EMB), lambda i: (0, 0)),
            pl.BlockSpec((1, EMB), lambda i: (0, 0)),
            pl.BlockSpec((1, EMB), lambda i: (0, 0)),
        ],
        out_specs=pl.BlockSpec((TC_BLK // 128, 128), lambda i: (i, 0)),
        out_shape=jax.ShapeDtypeStruct((E_ROWS, 128), jnp.float32),
        compiler_params=pltpu.CompilerParams(
            dimension_semantics=("arbitrary",),
        ),
    )(x, w, b_row, u_row)


_MESH = plsc.VectorSubcoreMesh(core_axis_name="c", subcore_axis_name="s")


def _sc1_body(x_hbm, ids_hbm, e_hbm, p_hbm, dpart_hbm,
              shared_p, xbuf, wbuf, zbuf, ebuf, idbuf, dtile):
    c = lax.axis_index("c")
    s = lax.axis_index("s")
    tile = c * NS + s
    row_base = tile * ROWS_PER_TILE
    seg_base = s * SEG_PER_TILE
    z16 = jnp.zeros((LANES,), jnp.float32)

    @pl.loop(0, DN_PAD // LANES)
    def _(r):
        dtile[pl.ds(r * LANES, LANES)] = z16

    @pl.loop(0, WB)
    def _(r):
        for f in range(EMB // LANES):
            zbuf[r, pl.ds(f * LANES, LANES)] = z16

    @pl.loop(0, SEG_PER_TILE // WB)
    def _(k):
        pltpu.sync_copy(zbuf, shared_p.at[pl.ds(seg_base + k * WB, WB)])

    plsc.subcore_barrier()

    @pl.loop(0, NCHUNKS)
    def _(k):
        base = row_base + k * CHUNK
        pltpu.sync_copy(x_hbm.at[pl.ds(base, CHUNK)], xbuf)
        pltpu.sync_copy(e_hbm.at[pl.ds(base, CHUNK)], ebuf.at[pl.ds(0, CHUNK)])
        pltpu.sync_copy(ids_hbm.at[pl.ds(base, CHUNK)], idbuf)
        @pl.loop(0, CHUNK, unroll=8)
        def _(row):
            bvec = plsc.load_gather(
                ebuf, [jnp.full((LANES,), row, jnp.int32)])
            for f in range(EMB // LANES):
                sl = pl.ds(f * LANES, LANES)
                wbuf[row, sl] = xbuf[row, sl] * bvec
        for g in range(CHUNK // LANES):
            sl = pl.ds(g * LANES, LANES)
            plsc.addupdate_scatter(dtile, [idbuf[sl]], ebuf[sl])
        pltpu.sync_copy(wbuf, shared_p.at[idbuf], add=True)

    plsc.subcore_barrier()

    out_off = c * SEG_PAD + seg_base

    @pl.loop(0, SEG_PER_TILE // WB)
    def _(j):
        pltpu.sync_copy(shared_p.at[pl.ds(seg_base + j * WB, WB)],
                        p_hbm.at[pl.ds(out_off + j * WB, WB)])

    pltpu.sync_copy(dtile, dpart_hbm.at[pl.ds(tile * DN_PAD, DN_PAD)])


def _sc1(x, ids, e_flat):
    f = pl.kernel(
        _sc1_body,
        out_type=[
            jax.ShapeDtypeStruct((NC * SEG_PAD, EMB), jnp.float32),
            jax.ShapeDtypeStruct((NW * DN_PAD,), jnp.float32),
        ],
        mesh=_MESH,
        compiler_params=pltpu.CompilerParams(needs_layout_passes=False),
        scratch_types=[
            pltpu.VMEM_SHARED((SEG_PAD, EMB), jnp.float32),
            pltpu.VMEM((CHUNK, EMB), jnp.float32),
            pltpu.VMEM((CHUNK, EMB), jnp.float32),
            pltpu.VMEM((WB, EMB), jnp.float32),
            pltpu.VMEM((128,), jnp.float32),
            pltpu.VMEM((CHUNK,), jnp.int32),
            pltpu.VMEM((DN_PAD,), jnp.float32),
        ],
    )
    return f(x, ids, e_flat)


def _sc2_body(p_hbm, dpart_hbm, e_hbm, ids_hbm, out_hbm, alpha_hbm,
              shared_den, p0buf, p1buf, denbuf, tslice, ebuf, idbuf, abuf):
    c = lax.axis_index("c")
    s = lax.axis_index("s")
    tile = c * NS + s

    dn_base = s * DN_SLICE

    @pl.loop(0, DN_SLICE // LANES)
    def _(j):
        denbuf[pl.ds(dn_base + j * LANES, LANES)] = jnp.zeros(
            (LANES,), jnp.float32)

    @pl.loop(0, NW)
    def _(t):
        pltpu.sync_copy(dpart_hbm.at[pl.ds(t * DN_PAD + dn_base, DN_SLICE)],
                        tslice)
        for j in range(DN_SLICE // LANES):
            sl = pl.ds(j * LANES, LANES)
            dsl = pl.ds(dn_base + j * LANES, LANES)
            denbuf[dsl] = denbuf[dsl] + tslice[sl]

    pltpu.sync_copy(denbuf.at[pl.ds(dn_base, DN_SLICE)],
                    shared_den.at[pl.ds(dn_base, DN_SLICE)])
    plsc.subcore_barrier()
    pltpu.sync_copy(shared_den, denbuf)

    out_base = tile * OUT_PER_TILE
    pltpu.sync_copy(p_hbm.at[pl.ds(out_base, OUT_PER_TILE)], p0buf)
    pltpu.sync_copy(p_hbm.at[pl.ds(SEG_PAD + out_base, OUT_PER_TILE)], p1buf)

    @pl.loop(0, OUT_PER_TILE)
    def _(r):
        dvec = plsc.load_gather(
            denbuf, [jnp.full((LANES,), out_base, jnp.int32) + r])
        inv = 1.0 / jnp.where(dvec > 0.0, dvec, 1.0)
        for f in range(EMB // LANES):
            sl = pl.ds(f * LANES, LANES)
            p0buf[r, sl] = (p0buf[r, sl] + p1buf[r, sl]) * inv

    pltpu.sync_copy(p0buf, out_hbm.at[pl.ds(out_base, OUT_PER_TILE)])

    row_base = tile * ROWS_PER_TILE

    @pl.loop(0, A_NCHUNKS)
    def _(k):
        base = row_base + k * A_CHUNK
        pltpu.sync_copy(e_hbm.at[pl.ds(base, A_CHUNK)], ebuf)
        pltpu.sync_copy(ids_hbm.at[pl.ds(base, A_CHUNK)], idbuf)
        for g in range(A_CHUNK // LANES):
            sl = pl.ds(g * LANES, LANES)
            dvec = plsc.load_gather(denbuf, [idbuf[sl]])
            abuf[sl] = ebuf[sl] / dvec
        pltpu.sync_copy(abuf, alpha_hbm.at[pl.ds(base, A_CHUNK)])


def _sc2(p, dpart, e_flat, ids):
    f = pl.kernel(
        _sc2_body,
        out_type=[
            jax.ShapeDtypeStruct((SEG_PAD, EMB), jnp.float32),
            jax.ShapeDtypeStruct((N,), jnp.float32),
        ],
        mesh=_MESH,
        compiler_params=pltpu.CompilerParams(needs_layout_passes=False),
        scratch_types=[
            pltpu.VMEM_SHARED((DN_PAD,), jnp.float32),
            pltpu.VMEM((OUT_PER_TILE, EMB), jnp.float32),
            pltpu.VMEM((OUT_PER_TILE, EMB), jnp.float32),
            pltpu.VMEM((DN_PAD,), jnp.float32),
            pltpu.VMEM((DN_SLICE,), jnp.float32),
            pltpu.VMEM((A_CHUNK,), jnp.float32),
            pltpu.VMEM((A_CHUNK,), jnp.int32),
            pltpu.VMEM((A_CHUNK,), jnp.float32),
        ],
    )
    return f(p, dpart, e_flat, ids)


@jax.jit
def kernel(x, type_batch_idx, w_omega, b_omega, u_omega):
    b_row = b_omega.reshape(1, EMB)
    u_row = u_omega.reshape(1, EMB)
    e2d = _compute_e(x, w_omega, b_row, u_row)
    e_flat = e2d.reshape(E_ROWS * 128)
    p, dpart = _sc1(x, type_batch_idx, e_flat)
    out_pad, alphas = _sc2(p, dpart, e_flat, type_batch_idx)
    return out_pad[:NSEG], alphas.reshape(N, 1)
